# baseline (speedup 1.0000x reference)
import functools

import jax
import jax.numpy as jnp
from jax import lax
from jax.experimental import pallas as pl
from jax.experimental.pallas import tpu as pltpu
from jax.experimental.pallas import tpu_sc as plsc

N_BUF = 3
LANES = 128


@functools.lru_cache(maxsize=None)
def _make_gather(batch: int, hist: int, dim: int):
    info = plsc.get_sparse_core_info()
    num_cores, num_subcores = info.num_cores, info.num_subcores
    num_workers = num_cores * num_subcores
    assert batch % num_workers == 0
    per_worker = batch // num_workers
    hist_pad = (hist + 7) // 8 * 8
    assert 2 * dim == LANES
    assert per_worker >= 6 and (per_worker - 5) % 3 == 0
    n_groups = (per_worker - 5) // 3

    mesh = plsc.VectorSubcoreMesh(core_axis_name="c", subcore_axis_name="s")

    @functools.partial(
        pl.kernel,
        mesh=mesh,
        out_type=jax.ShapeDtypeStruct((batch, hist_pad, LANES), jnp.float32),
        scratch_types=[
            pltpu.VMEM((per_worker, LANES), jnp.int32),
            [pltpu.VMEM((hist_pad, LANES), jnp.float32)] * N_BUF,
            [pltpu.SemaphoreType.DMA] * N_BUF,
            [pltpu.SemaphoreType.DMA] * N_BUF,
        ],
    )
    def gather(idx_hbm, tab_hbm, out_hbm, idx_v, bufs, sem_g, sem_w):
        wid = lax.axis_index("s") * num_cores + lax.axis_index("c")
        base_w = pl.multiple_of(wid * per_worker, 8)
        pltpu.sync_copy(idx_hbm.at[pl.ds(base_w, per_worker)], idx_v)

        def start_g(j, b):
            idx = idx_v.at[j, pl.ds(0, hist)]
            pltpu.async_copy(tab_hbm.at[idx], bufs[b].at[pl.ds(0, hist)],
                             sem_g[b])

        def wait_g(b):
            idx = idx_v.at[0, pl.ds(0, hist)]
            pltpu.make_async_copy(tab_hbm.at[idx],
                                  bufs[b].at[pl.ds(0, hist)], sem_g[b]).wait()

        def start_w(j, b):
            pltpu.async_copy(bufs[b], out_hbm.at[base_w + j], sem_w[b])

        def wait_w(b):
            pltpu.make_async_copy(bufs[b], out_hbm.at[0], sem_w[b]).wait()

        start_g(0, 0)
        start_g(1, 1)
        wait_g(0)
        start_w(0, 0)
        start_g(2, 2)
        wait_g(1)
        start_w(1, 1)
        wait_w(0)
        start_g(3, 0)
        wait_g(2)
        start_w(2, 2)
        wait_w(1)
        start_g(4, 1)
        wait_g(0)
        start_w(3, 0)

        def body(g, carry):
            for k in range(3):
                i = 4 + 3 * g + k
                b = (1 + k) % 3
                b_next = (2 + k) % 3
                wait_w(b_next)
                start_g(i + 1, b_next)
                wait_g(b)
                start_w(i, b)
            return carry

        lax.fori_loop(0, n_groups, body, 0)

        wait_g(1)
        start_w(per_worker - 1, 1)
        wait_w(2)
        wait_w(0)
        wait_w(1)

    return gather


@functools.lru_cache(maxsize=None)
def _make_split(batch: int, hist: int, dim: int, block: int):
    hist_pad = (hist + 7) // 8 * 8

    def split_body(fused_ref, amp_ref, ph_ref):
        x = fused_ref[...]
        amp_ref[...] = x[:, :hist, :dim]
        ph_ref[...] = x[:, :hist, dim:]

    return pl.pallas_call(
        split_body,
        grid=(batch // block,),
        in_specs=[
            pl.BlockSpec((block, hist_pad, LANES), lambda i: (i, 0, 0)),
        ],
        out_specs=(
            pl.BlockSpec((block, hist, dim), lambda i: (i, 0, 0)),
            pl.BlockSpec((block, hist, dim), lambda i: (i, 0, 0)),
        ),
        out_shape=(
            jax.ShapeDtypeStruct((batch, hist, dim), jnp.float32),
            jax.ShapeDtypeStruct((batch, hist, dim), jnp.float32),
        ),
        compiler_params=pltpu.CompilerParams(
            dimension_semantics=("arbitrary",),
        ),
    )


def kernel(indices, amplitude_table, phase_table):
    batch, hist = indices.shape
    dim = amplitude_table.shape[1]
    idx_pad = jnp.pad(indices, ((0, 0), (0, LANES - hist)))
    fused_table = jnp.concatenate([amplitude_table, phase_table], axis=1)
    fused = _make_gather(batch, hist, dim)(idx_pad, fused_table)
    amp = fused[:, :hist, :dim]
    ph = fused[:, :hist, dim:]
    return amp, ph


# speedup vs baseline: 2.0601x; 2.0601x over previous
import functools

import jax
import jax.numpy as jnp
from jax import lax
from jax.experimental import pallas as pl
from jax.experimental.pallas import tpu as pltpu
from jax.experimental.pallas import tpu_sc as plsc

N_BUF = 3
LANES = 128


@functools.lru_cache(maxsize=None)
def _make_gather(batch: int, hist: int, dim: int):
    info = plsc.get_sparse_core_info()
    num_cores, num_subcores = info.num_cores, info.num_subcores
    num_workers = num_cores * num_subcores
    assert batch % num_workers == 0
    per_worker = batch // num_workers
    hist_pad = (hist + 7) // 8 * 8
    assert 2 * dim == LANES
    assert per_worker >= 6 and (per_worker - 5) % 3 == 0
    n_groups = (per_worker - 5) // 3

    mesh = plsc.VectorSubcoreMesh(core_axis_name="c", subcore_axis_name="s")

    @functools.partial(
        pl.kernel,
        mesh=mesh,
        out_type=jax.ShapeDtypeStruct((batch, hist_pad, LANES), jnp.float32),
        scratch_types=[
            pltpu.VMEM((per_worker, LANES), jnp.int32),
            [pltpu.VMEM((hist_pad, LANES), jnp.float32)] * N_BUF,
            [pltpu.SemaphoreType.DMA] * N_BUF,
            [pltpu.SemaphoreType.DMA] * N_BUF,
        ],
    )
    def gather(idx_hbm, tab_hbm, out_hbm, idx_v, bufs, sem_g, sem_w):
        wid = lax.axis_index("s") * num_cores + lax.axis_index("c")
        base_w = pl.multiple_of(wid * per_worker, 8)
        pltpu.sync_copy(idx_hbm.at[pl.ds(base_w, per_worker)], idx_v)

        def start_g(j, b):
            idx = idx_v.at[j, pl.ds(0, hist)]
            pltpu.async_copy(tab_hbm.at[idx], bufs[b].at[pl.ds(0, hist)],
                             sem_g[b])

        def wait_g(b):
            idx = idx_v.at[0, pl.ds(0, hist)]
            pltpu.make_async_copy(tab_hbm.at[idx],
                                  bufs[b].at[pl.ds(0, hist)], sem_g[b]).wait()

        def start_w(j, b):
            pltpu.async_copy(bufs[b], out_hbm.at[base_w + j], sem_w[b])

        def wait_w(b):
            pltpu.make_async_copy(bufs[b], out_hbm.at[0], sem_w[b]).wait()

        start_g(0, 0)
        start_g(1, 1)
        wait_g(0)
        start_w(0, 0)
        start_g(2, 2)
        wait_g(1)
        start_w(1, 1)
        wait_w(0)
        start_g(3, 0)
        wait_g(2)
        start_w(2, 2)
        wait_w(1)
        start_g(4, 1)
        wait_g(0)
        start_w(3, 0)

        def body(g, carry):
            for k in range(3):
                i = 4 + 3 * g + k
                b = (1 + k) % 3
                b_next = (2 + k) % 3
                wait_w(b_next)
                start_g(i + 1, b_next)
                wait_g(b)
                start_w(i, b)
            return carry

        lax.fori_loop(0, n_groups, body, 0)

        wait_g(1)
        start_w(per_worker - 1, 1)
        wait_w(2)
        wait_w(0)
        wait_w(1)

    return gather


@functools.lru_cache(maxsize=None)
def _make_split(batch: int, hist: int, dim: int, block: int):
    hist_pad = (hist + 7) // 8 * 8

    def split_body(fused_ref, amp_ref, ph_ref):
        for h in range(hist):
            t = jnp.transpose(fused_ref[:, h, :])
            amp_ref[h] = t[:dim]
            ph_ref[h] = t[dim:]

    return pl.pallas_call(
        split_body,
        grid=(batch // block,),
        in_specs=[
            pl.BlockSpec((block, hist_pad, LANES), lambda i: (i, 0, 0)),
        ],
        out_specs=(
            pl.BlockSpec((hist, dim, block), lambda i: (0, 0, i)),
            pl.BlockSpec((hist, dim, block), lambda i: (0, 0, i)),
        ),
        out_shape=(
            jax.ShapeDtypeStruct((hist, dim, batch), jnp.float32),
            jax.ShapeDtypeStruct((hist, dim, batch), jnp.float32),
        ),
        compiler_params=pltpu.CompilerParams(
            dimension_semantics=("arbitrary",),
        ),
    )


def kernel(indices, amplitude_table, phase_table):
    batch, hist = indices.shape
    dim = amplitude_table.shape[1]
    idx_pad = jnp.pad(indices, ((0, 0), (0, LANES - hist)))
    fused_table = jnp.concatenate([amplitude_table, phase_table], axis=1)
    fused = _make_gather(batch, hist, dim)(idx_pad, fused_table)
    amp_t, ph_t = _make_split(batch, hist, dim, 128)(fused)
    return jnp.transpose(amp_t, (2, 0, 1)), jnp.transpose(ph_t, (2, 0, 1))


# speedup vs baseline: 2.1639x; 1.0504x over previous
import functools

import jax
import jax.numpy as jnp
from jax import lax
from jax.experimental import pallas as pl
from jax.experimental.pallas import tpu as pltpu
from jax.experimental.pallas import tpu_sc as plsc

N_BUF = 3
LANES = 128


@functools.lru_cache(maxsize=None)
def _make_gather(batch: int, hist: int, dim: int):
    info = plsc.get_sparse_core_info()
    num_cores, num_subcores = info.num_cores, info.num_subcores
    num_workers = num_cores * num_subcores
    assert batch % num_workers == 0
    per_worker = batch // num_workers
    hist_pad = (hist + 7) // 8 * 8
    assert 2 * dim == LANES
    assert per_worker >= 6 and (per_worker - 5) % 3 == 0
    n_groups = (per_worker - 5) // 3

    mesh = plsc.VectorSubcoreMesh(core_axis_name="c", subcore_axis_name="s")

    @functools.partial(
        pl.kernel,
        mesh=mesh,
        out_type=jax.ShapeDtypeStruct((batch, hist_pad, LANES), jnp.float32),
        scratch_types=[
            pltpu.VMEM((per_worker, LANES), jnp.int32),
            [pltpu.VMEM((hist_pad, LANES), jnp.float32)] * N_BUF,
            [pltpu.SemaphoreType.DMA] * N_BUF,
            [pltpu.SemaphoreType.DMA] * N_BUF,
        ],
    )
    def gather(idx_hbm, tab_hbm, out_hbm, idx_v, bufs, sem_g, sem_w):
        wid = lax.axis_index("s") * num_cores + lax.axis_index("c")
        base_w = pl.multiple_of(wid * per_worker, 8)
        pltpu.sync_copy(idx_hbm.at[pl.ds(base_w, per_worker)], idx_v)

        def start_g(j, b):
            idx = idx_v.at[j, pl.ds(0, hist)]
            pltpu.async_copy(tab_hbm.at[idx], bufs[b].at[pl.ds(0, hist)],
                             sem_g[b])

        def wait_g(b):
            idx = idx_v.at[0, pl.ds(0, hist)]
            pltpu.make_async_copy(tab_hbm.at[idx],
                                  bufs[b].at[pl.ds(0, hist)], sem_g[b]).wait()

        def start_w(j, b):
            pltpu.async_copy(bufs[b], out_hbm.at[base_w + j], sem_w[b])

        def wait_w(b):
            pltpu.make_async_copy(bufs[b], out_hbm.at[0], sem_w[b]).wait()

        start_g(0, 0)
        start_g(1, 1)
        wait_g(0)
        start_w(0, 0)
        start_g(2, 2)
        wait_g(1)
        start_w(1, 1)
        wait_w(0)
        start_g(3, 0)
        wait_g(2)
        start_w(2, 2)
        wait_w(1)
        start_g(4, 1)
        wait_g(0)
        start_w(3, 0)

        def body(g, carry):
            for k in range(3):
                i = 4 + 3 * g + k
                b = (1 + k) % 3
                b_next = (2 + k) % 3
                wait_w(b_next)
                start_g(i + 1, b_next)
                wait_g(b)
                start_w(i, b)
            return carry

        lax.fori_loop(0, n_groups, body, 0)

        wait_g(1)
        start_w(per_worker - 1, 1)
        wait_w(2)
        wait_w(0)
        wait_w(1)

    return gather


@functools.lru_cache(maxsize=None)
def _make_split(batch: int, hist: int, dim: int, block: int):
    hist_pad = (hist + 7) // 8 * 8

    def split_body(fused_ref, amp_ref, ph_ref):
        for h in range(hist):
            t = jnp.transpose(fused_ref[:, h, :])
            amp_ref[h] = t[:dim]
            ph_ref[h] = t[dim:]

    return pl.pallas_call(
        split_body,
        grid=(batch // block,),
        in_specs=[
            pl.BlockSpec((block, hist_pad, LANES), lambda i: (i, 0, 0)),
        ],
        out_specs=(
            pl.BlockSpec((hist, dim, block), lambda i: (0, 0, i)),
            pl.BlockSpec((hist, dim, block), lambda i: (0, 0, i)),
        ),
        out_shape=(
            jax.ShapeDtypeStruct((hist, dim, batch), jnp.float32),
            jax.ShapeDtypeStruct((hist, dim, batch), jnp.float32),
        ),
        compiler_params=pltpu.CompilerParams(
            dimension_semantics=("arbitrary",),
        ),
    )


def kernel(indices, amplitude_table, phase_table):
    batch, hist = indices.shape
    dim = amplitude_table.shape[1]
    idx_pad = jnp.pad(indices, ((0, 0), (0, LANES - hist)))
    fused_table = jnp.concatenate([amplitude_table, phase_table], axis=1)
    fused = _make_gather(batch, hist, dim)(idx_pad, fused_table)
    amp_t, ph_t = _make_split(batch, hist, dim, 256)(fused)
    return jnp.transpose(amp_t, (2, 0, 1)), jnp.transpose(ph_t, (2, 0, 1))


# speedup vs baseline: 2.1975x; 1.0155x over previous
import functools

import jax
import jax.numpy as jnp
from jax import lax
from jax.experimental import pallas as pl
from jax.experimental.pallas import tpu as pltpu
from jax.experimental.pallas import tpu_sc as plsc

N_BUF = 3
LANES = 128


@functools.lru_cache(maxsize=None)
def _make_gather(batch: int, hist: int, dim: int):
    info = plsc.get_sparse_core_info()
    num_cores, num_subcores = info.num_cores, info.num_subcores
    num_workers = num_cores * num_subcores
    assert batch % num_workers == 0
    per_worker = batch // num_workers
    hist_pad = (hist + 7) // 8 * 8
    assert 2 * dim == LANES
    assert per_worker >= 6 and (per_worker - 5) % 3 == 0
    n_groups = (per_worker - 5) // 3

    mesh = plsc.VectorSubcoreMesh(core_axis_name="c", subcore_axis_name="s")

    @functools.partial(
        pl.kernel,
        mesh=mesh,
        out_type=jax.ShapeDtypeStruct((batch, hist_pad, LANES), jnp.float32),
        scratch_types=[
            pltpu.VMEM((per_worker, LANES), jnp.int32),
            [pltpu.VMEM((hist_pad, LANES), jnp.float32)] * N_BUF,
            [pltpu.SemaphoreType.DMA] * N_BUF,
            [pltpu.SemaphoreType.DMA] * N_BUF,
        ],
    )
    def gather(idx_hbm, tab_hbm, out_hbm, idx_v, bufs, sem_g, sem_w):
        wid = lax.axis_index("s") * num_cores + lax.axis_index("c")
        base_w = pl.multiple_of(wid * per_worker, 8)
        pltpu.sync_copy(idx_hbm.at[pl.ds(base_w, per_worker)], idx_v)

        def start_g(j, b):
            idx = idx_v.at[j, pl.ds(0, hist)]
            pltpu.async_copy(tab_hbm.at[idx], bufs[b].at[pl.ds(0, hist)],
                             sem_g[b])

        def wait_g(b):
            idx = idx_v.at[0, pl.ds(0, hist)]
            pltpu.make_async_copy(tab_hbm.at[idx],
                                  bufs[b].at[pl.ds(0, hist)], sem_g[b]).wait()

        def start_w(j, b):
            pltpu.async_copy(bufs[b], out_hbm.at[base_w + j], sem_w[b])

        def wait_w(b):
            pltpu.make_async_copy(bufs[b], out_hbm.at[0], sem_w[b]).wait()

        start_g(0, 0)
        start_g(1, 1)
        wait_g(0)
        start_w(0, 0)
        start_g(2, 2)
        wait_g(1)
        start_w(1, 1)
        wait_w(0)
        start_g(3, 0)
        wait_g(2)
        start_w(2, 2)
        wait_w(1)
        start_g(4, 1)
        wait_g(0)
        start_w(3, 0)

        def body(g, carry):
            for k in range(3):
                i = 4 + 3 * g + k
                b = (1 + k) % 3
                b_next = (2 + k) % 3
                wait_w(b_next)
                start_g(i + 1, b_next)
                wait_g(b)
                start_w(i, b)
            return carry

        lax.fori_loop(0, n_groups, body, 0)

        wait_g(1)
        start_w(per_worker - 1, 1)
        wait_w(2)
        wait_w(0)
        wait_w(1)

    return gather


@functools.lru_cache(maxsize=None)
def _make_split(batch: int, hist: int, dim: int, block: int):
    hist_pad = (hist + 7) // 8 * 8

    def split_body(fused_ref, amp_ref, ph_ref):
        for h in range(hist):
            t = jnp.transpose(fused_ref[:, h, :])
            amp_ref[h] = t[:dim]
            ph_ref[h] = t[dim:]

    return pl.pallas_call(
        split_body,
        grid=(batch // block,),
        in_specs=[
            pl.BlockSpec((block, hist_pad, LANES), lambda i: (i, 0, 0)),
        ],
        out_specs=(
            pl.BlockSpec((hist, dim, block), lambda i: (0, 0, i)),
            pl.BlockSpec((hist, dim, block), lambda i: (0, 0, i)),
        ),
        out_shape=(
            jax.ShapeDtypeStruct((hist, dim, batch), jnp.float32),
            jax.ShapeDtypeStruct((hist, dim, batch), jnp.float32),
        ),
        compiler_params=pltpu.CompilerParams(
            dimension_semantics=("arbitrary",),
        ),
    )


def kernel(indices, amplitude_table, phase_table):
    batch, hist = indices.shape
    dim = amplitude_table.shape[1]
    idx_pad = jnp.pad(indices, ((0, 0), (0, LANES - hist)))
    fused_table = jnp.concatenate([amplitude_table, phase_table], axis=1)
    fused = _make_gather(batch, hist, dim)(idx_pad, fused_table)
    amp_t, ph_t = _make_split(batch, hist, dim, 512)(fused)
    return jnp.transpose(amp_t, (2, 0, 1)), jnp.transpose(ph_t, (2, 0, 1))


# speedup vs baseline: 2.3048x; 1.0488x over previous
import functools

import jax
import jax.numpy as jnp
from jax import lax
from jax.experimental import pallas as pl
from jax.experimental.pallas import tpu as pltpu
from jax.experimental.pallas import tpu_sc as plsc

N_BUF = 3
LANES = 128


K_EL = 2


@functools.lru_cache(maxsize=None)
def _make_gather(batch: int, hist: int, dim: int):
    info = plsc.get_sparse_core_info()
    num_cores, num_subcores = info.num_cores, info.num_subcores
    num_workers = num_cores * num_subcores
    assert batch % (num_workers * K_EL) == 0
    per_worker = batch // num_workers
    n_chunks = per_worker // K_EL
    hist_pad = (hist + 7) // 8 * 8
    assert 2 * dim == LANES
    assert n_chunks >= 6
    n_groups = (n_chunks - 5) // 3
    leftover = (n_chunks - 5) % 3

    mesh = plsc.VectorSubcoreMesh(core_axis_name="c", subcore_axis_name="s")

    @functools.partial(
        pl.kernel,
        mesh=mesh,
        out_type=jax.ShapeDtypeStruct((batch, hist_pad, LANES), jnp.float32),
        scratch_types=[
            pltpu.VMEM((per_worker, LANES), jnp.int32),
            [pltpu.VMEM((K_EL, hist_pad, LANES), jnp.float32)] * N_BUF,
            [pltpu.SemaphoreType.DMA] * N_BUF,
            [pltpu.SemaphoreType.DMA] * N_BUF,
        ],
    )
    def gather(idx_hbm, tab_hbm, out_hbm, idx_v, bufs, sem_g, sem_w):
        wid = lax.axis_index("s") * num_cores + lax.axis_index("c")
        base_w = pl.multiple_of(wid * per_worker, 8)
        pltpu.sync_copy(idx_hbm.at[pl.ds(base_w, per_worker)], idx_v)

        def start_g(j, b):
            for t in range(K_EL):
                idx = idx_v.at[j * K_EL + t, pl.ds(0, hist)]
                pltpu.async_copy(tab_hbm.at[idx],
                                 bufs[b].at[t, pl.ds(0, hist)], sem_g[b])

        def wait_g(b):
            idx = idx_v.at[0, pl.ds(0, hist)]
            for t in range(K_EL):
                pltpu.make_async_copy(tab_hbm.at[idx],
                                      bufs[b].at[t, pl.ds(0, hist)],
                                      sem_g[b]).wait()

        def start_w(j, b):
            dst = out_hbm.at[pl.ds(base_w + j * K_EL, K_EL)]
            pltpu.async_copy(bufs[b], dst, sem_w[b])

        def wait_w(b):
            dst = out_hbm.at[pl.ds(0, K_EL)]
            pltpu.make_async_copy(bufs[b], dst, sem_w[b]).wait()

        def steady(i, b, b_next):
            wait_w(b_next)
            start_g(i + 1, b_next)
            wait_g(b)
            start_w(i, b)

        start_g(0, 0)
        start_g(1, 1)
        wait_g(0)
        start_w(0, 0)
        start_g(2, 2)
        wait_g(1)
        start_w(1, 1)
        wait_w(0)
        start_g(3, 0)
        wait_g(2)
        start_w(2, 2)
        wait_w(1)
        start_g(4, 1)
        wait_g(0)
        start_w(3, 0)

        def body(g, carry):
            for k in range(3):
                i = 4 + 3 * g + k
                steady(i, (1 + k) % 3, (2 + k) % 3)
            return carry

        lax.fori_loop(0, n_groups, body, 0)

        first_left = 4 + 3 * n_groups
        for i in range(first_left, first_left + leftover):
            steady(i, i % 3, (i + 1) % 3)
        last = n_chunks - 1
        wait_g(last % 3)
        start_w(last, last % 3)
        for b in range(N_BUF):
            wait_w(b)

    return gather


@functools.lru_cache(maxsize=None)
def _make_split(batch: int, hist: int, dim: int, block: int):
    hist_pad = (hist + 7) // 8 * 8

    def split_body(fused_ref, amp_ref, ph_ref):
        for h in range(hist):
            t = jnp.transpose(fused_ref[:, h, :])
            amp_ref[h] = t[:dim]
            ph_ref[h] = t[dim:]

    return pl.pallas_call(
        split_body,
        grid=(batch // block,),
        in_specs=[
            pl.BlockSpec((block, hist_pad, LANES), lambda i: (i, 0, 0)),
        ],
        out_specs=(
            pl.BlockSpec((hist, dim, block), lambda i: (0, 0, i)),
            pl.BlockSpec((hist, dim, block), lambda i: (0, 0, i)),
        ),
        out_shape=(
            jax.ShapeDtypeStruct((hist, dim, batch), jnp.float32),
            jax.ShapeDtypeStruct((hist, dim, batch), jnp.float32),
        ),
        compiler_params=pltpu.CompilerParams(
            dimension_semantics=("arbitrary",),
        ),
    )


def kernel(indices, amplitude_table, phase_table):
    batch, hist = indices.shape
    dim = amplitude_table.shape[1]
    idx_pad = jnp.pad(indices, ((0, 0), (0, LANES - hist)))
    fused_table = jnp.concatenate([amplitude_table, phase_table], axis=1)
    fused = _make_gather(batch, hist, dim)(idx_pad, fused_table)
    amp_t, ph_t = _make_split(batch, hist, dim, 512)(fused)
    return jnp.transpose(amp_t, (2, 0, 1)), jnp.transpose(ph_t, (2, 0, 1))


# speedup vs baseline: 2.3197x; 1.0065x over previous
import functools

import jax
import jax.numpy as jnp
from jax import lax
from jax.experimental import pallas as pl
from jax.experimental.pallas import tpu as pltpu
from jax.experimental.pallas import tpu_sc as plsc

N_BUF = 3
LANES = 128


K_EL = 2


@functools.lru_cache(maxsize=None)
def _make_gather(batch: int, hist: int, dim: int, offset: int = 0,
                 idx_batch: int | None = None):
    if idx_batch is None:
        idx_batch = batch
    info = plsc.get_sparse_core_info()
    num_cores, num_subcores = info.num_cores, info.num_subcores
    num_workers = num_cores * num_subcores
    assert batch % (num_workers * K_EL) == 0
    per_worker = batch // num_workers
    n_chunks = per_worker // K_EL
    hist_pad = (hist + 7) // 8 * 8
    assert 2 * dim == LANES
    assert n_chunks >= 6
    n_groups = (n_chunks - 5) // 3
    leftover = (n_chunks - 5) % 3

    mesh = plsc.VectorSubcoreMesh(core_axis_name="c", subcore_axis_name="s")

    @functools.partial(
        pl.kernel,
        mesh=mesh,
        out_type=jax.ShapeDtypeStruct((batch, hist_pad, LANES), jnp.float32),
        scratch_types=[
            pltpu.VMEM((per_worker, LANES), jnp.int32),
            [pltpu.VMEM((K_EL, hist_pad, LANES), jnp.float32)] * N_BUF,
            [pltpu.SemaphoreType.DMA] * N_BUF,
            [pltpu.SemaphoreType.DMA] * N_BUF,
        ],
    )
    def gather(idx_hbm, tab_hbm, out_hbm, idx_v, bufs, sem_g, sem_w):
        wid = lax.axis_index("s") * num_cores + lax.axis_index("c")
        base_w = pl.multiple_of(wid * per_worker, 8)
        pltpu.sync_copy(idx_hbm.at[pl.ds(offset + base_w, per_worker)], idx_v)

        def start_g(j, b):
            for t in range(K_EL):
                idx = idx_v.at[j * K_EL + t, pl.ds(0, hist)]
                pltpu.async_copy(tab_hbm.at[idx],
                                 bufs[b].at[t, pl.ds(0, hist)], sem_g[b])

        def wait_g(b):
            idx = idx_v.at[0, pl.ds(0, hist)]
            for t in range(K_EL):
                pltpu.make_async_copy(tab_hbm.at[idx],
                                      bufs[b].at[t, pl.ds(0, hist)],
                                      sem_g[b]).wait()

        def start_w(j, b):
            dst = out_hbm.at[pl.ds(base_w + j * K_EL, K_EL)]
            pltpu.async_copy(bufs[b], dst, sem_w[b])

        def wait_w(b):
            dst = out_hbm.at[pl.ds(0, K_EL)]
            pltpu.make_async_copy(bufs[b], dst, sem_w[b]).wait()

        def steady(i, b, b_next):
            wait_w(b_next)
            start_g(i + 1, b_next)
            wait_g(b)
            start_w(i, b)

        start_g(0, 0)
        start_g(1, 1)
        wait_g(0)
        start_w(0, 0)
        start_g(2, 2)
        wait_g(1)
        start_w(1, 1)
        wait_w(0)
        start_g(3, 0)
        wait_g(2)
        start_w(2, 2)
        wait_w(1)
        start_g(4, 1)
        wait_g(0)
        start_w(3, 0)

        def body(g, carry):
            for k in range(3):
                i = 4 + 3 * g + k
                steady(i, (1 + k) % 3, (2 + k) % 3)
            return carry

        lax.fori_loop(0, n_groups, body, 0)

        first_left = 4 + 3 * n_groups
        for i in range(first_left, first_left + leftover):
            steady(i, i % 3, (i + 1) % 3)
        last = n_chunks - 1
        wait_g(last % 3)
        start_w(last, last % 3)
        for b in range(N_BUF):
            wait_w(b)

    return gather


@functools.lru_cache(maxsize=None)
def _make_split(batch: int, part: int, hist: int, dim: int, block: int,
                block_off: int, aliased: bool):
    hist_pad = (hist + 7) // 8 * 8

    def split_body(*refs):
        fused_ref = refs[-3]
        amp_ref, ph_ref = refs[-2], refs[-1]
        for h in range(hist):
            t = jnp.transpose(fused_ref[:, h, :])
            amp_ref[h] = t[:dim]
            ph_ref[h] = t[dim:]

    in_specs = [pl.BlockSpec((block, hist_pad, LANES), lambda i: (i, 0, 0))]
    if aliased:
        in_specs = [
            pl.BlockSpec(memory_space=pl.ANY),
            pl.BlockSpec(memory_space=pl.ANY),
        ] + in_specs

    return pl.pallas_call(
        split_body,
        grid=(part // block,),
        in_specs=in_specs,
        out_specs=(
            pl.BlockSpec((hist, dim, block),
                         lambda i: (0, 0, block_off + i)),
            pl.BlockSpec((hist, dim, block),
                         lambda i: (0, 0, block_off + i)),
        ),
        out_shape=(
            jax.ShapeDtypeStruct((hist, dim, batch), jnp.float32),
            jax.ShapeDtypeStruct((hist, dim, batch), jnp.float32),
        ),
        input_output_aliases={0: 0, 1: 1} if aliased else {},
        compiler_params=pltpu.CompilerParams(
            dimension_semantics=("arbitrary",),
        ),
    )


SPLIT_BLOCK = 512


def kernel(indices, amplitude_table, phase_table):
    batch, hist = indices.shape
    dim = amplitude_table.shape[1]
    half = batch // 2
    idx_pad = jnp.pad(indices, ((0, 0), (0, LANES - hist)))
    fused_table = jnp.concatenate([amplitude_table, phase_table], axis=1)
    fused0 = _make_gather(half, hist, dim, 0, batch)(idx_pad, fused_table)
    fused1 = _make_gather(half, hist, dim, half, batch)(idx_pad, fused_table)
    nb = half // SPLIT_BLOCK
    amp_t, ph_t = _make_split(batch, half, hist, dim, SPLIT_BLOCK,
                              0, False)(fused0)
    amp_t, ph_t = _make_split(batch, half, hist, dim, SPLIT_BLOCK,
                              nb, True)(amp_t, ph_t, fused1)
    return jnp.transpose(amp_t, (2, 0, 1)), jnp.transpose(ph_t, (2, 0, 1))


# speedup vs baseline: 2.3202x; 1.0002x over previous
import functools

import jax
import jax.numpy as jnp
from jax import lax
from jax.experimental import pallas as pl
from jax.experimental.pallas import tpu as pltpu
from jax.experimental.pallas import tpu_sc as plsc

N_BUF = 3
LANES = 128


K_EL = 2


@functools.lru_cache(maxsize=None)
def _make_gather(batch: int, hist: int, dim: int, offset: int = 0,
                 idx_batch: int | None = None):
    if idx_batch is None:
        idx_batch = batch
    info = plsc.get_sparse_core_info()
    num_cores, num_subcores = info.num_cores, info.num_subcores
    num_workers = num_cores * num_subcores
    assert batch % (num_workers * K_EL) == 0
    per_worker = batch // num_workers
    n_chunks = per_worker // K_EL
    hist_pad = (hist + 7) // 8 * 8
    assert 2 * dim == LANES
    assert n_chunks >= 6
    n_groups = (n_chunks - 5) // 3
    leftover = (n_chunks - 5) % 3

    mesh = plsc.VectorSubcoreMesh(core_axis_name="c", subcore_axis_name="s")

    @functools.partial(
        pl.kernel,
        mesh=mesh,
        out_type=jax.ShapeDtypeStruct((batch, hist_pad, LANES), jnp.float32),
        scratch_types=[
            pltpu.VMEM((per_worker, LANES), jnp.int32),
            [pltpu.VMEM((K_EL, hist_pad, LANES), jnp.float32)] * N_BUF,
            [pltpu.SemaphoreType.DMA] * N_BUF,
            [pltpu.SemaphoreType.DMA] * N_BUF,
        ],
    )
    def gather(idx_hbm, tab_hbm, out_hbm, idx_v, bufs, sem_g, sem_w):
        wid = lax.axis_index("s") * num_cores + lax.axis_index("c")
        base_w = pl.multiple_of(wid * per_worker, 8)
        pltpu.sync_copy(idx_hbm.at[pl.ds(offset + base_w, per_worker)], idx_v)

        def start_g(j, b):
            for t in range(K_EL):
                idx = idx_v.at[j * K_EL + t, pl.ds(0, hist)]
                pltpu.async_copy(tab_hbm.at[idx],
                                 bufs[b].at[t, pl.ds(0, hist)], sem_g[b])

        def wait_g(b):
            idx = idx_v.at[0, pl.ds(0, hist)]
            for t in range(K_EL):
                pltpu.make_async_copy(tab_hbm.at[idx],
                                      bufs[b].at[t, pl.ds(0, hist)],
                                      sem_g[b]).wait()

        def start_w(j, b):
            dst = out_hbm.at[pl.ds(base_w + j * K_EL, K_EL)]
            pltpu.async_copy(bufs[b], dst, sem_w[b])

        def wait_w(b):
            dst = out_hbm.at[pl.ds(0, K_EL)]
            pltpu.make_async_copy(bufs[b], dst, sem_w[b]).wait()

        def steady(i, b, b_next):
            wait_w(b_next)
            start_g(i + 1, b_next)
            wait_g(b)
            start_w(i, b)

        start_g(0, 0)
        start_g(1, 1)
        wait_g(0)
        start_w(0, 0)
        start_g(2, 2)
        wait_g(1)
        start_w(1, 1)
        wait_w(0)
        start_g(3, 0)
        wait_g(2)
        start_w(2, 2)
        wait_w(1)
        start_g(4, 1)
        wait_g(0)
        start_w(3, 0)

        def body(g, carry):
            for k in range(3):
                i = 4 + 3 * g + k
                steady(i, (1 + k) % 3, (2 + k) % 3)
            return carry

        lax.fori_loop(0, n_groups, body, 0)

        first_left = 4 + 3 * n_groups
        for i in range(first_left, first_left + leftover):
            steady(i, i % 3, (i + 1) % 3)
        last = n_chunks - 1
        wait_g(last % 3)
        start_w(last, last % 3)
        for b in range(N_BUF):
            wait_w(b)

    return gather


@functools.lru_cache(maxsize=None)
def _make_split(batch: int, part: int, hist: int, dim: int, block: int,
                block_off: int, aliased: bool):
    hist_pad = (hist + 7) // 8 * 8

    def split_body(*refs):
        fused_ref = refs[-3]
        amp_ref, ph_ref = refs[-2], refs[-1]
        for h in range(hist):
            t = jnp.transpose(fused_ref[:, h, :])
            amp_ref[h] = t[:dim]
            ph_ref[h] = t[dim:]

    in_specs = [pl.BlockSpec((block, hist_pad, LANES), lambda i: (i, 0, 0))]
    if aliased:
        in_specs = [
            pl.BlockSpec(memory_space=pl.ANY),
            pl.BlockSpec(memory_space=pl.ANY),
        ] + in_specs

    return pl.pallas_call(
        split_body,
        grid=(part // block,),
        in_specs=in_specs,
        out_specs=(
            pl.BlockSpec((hist, dim, block),
                         lambda i: (0, 0, block_off + i)),
            pl.BlockSpec((hist, dim, block),
                         lambda i: (0, 0, block_off + i)),
        ),
        out_shape=(
            jax.ShapeDtypeStruct((hist, dim, batch), jnp.float32),
            jax.ShapeDtypeStruct((hist, dim, batch), jnp.float32),
        ),
        input_output_aliases={0: 0, 1: 1} if aliased else {},
        compiler_params=pltpu.CompilerParams(
            dimension_semantics=("arbitrary",),
        ),
    )


SPLIT_BLOCK = 512


def kernel(indices, amplitude_table, phase_table):
    batch, hist = indices.shape
    dim = amplitude_table.shape[1]
    nsplit = 4
    part = batch // nsplit
    idx_pad = jnp.pad(indices, ((0, 0), (0, LANES - hist)))
    fused_table = jnp.concatenate([amplitude_table, phase_table], axis=1)
    nb = part // SPLIT_BLOCK
    amp_t = ph_t = None
    for k in range(nsplit):
        fused = _make_gather(part, hist, dim, k * part, batch)(
            idx_pad, fused_table)
        split = _make_split(batch, part, hist, dim, SPLIT_BLOCK,
                            k * nb, k > 0)
        if k == 0:
            amp_t, ph_t = split(fused)
        else:
            amp_t, ph_t = split(amp_t, ph_t, fused)
    return jnp.transpose(amp_t, (2, 0, 1)), jnp.transpose(ph_t, (2, 0, 1))
